# Initial kernel scaffold; baseline (speedup 1.0000x reference)
import jax
import jax.numpy as jnp
from jax.experimental import pallas as pl


def kernel(x, selector_weights, expert_weights, compressor_W, compressed_W):
    raise NotImplementedError("write your pallas kernel here")


# speedup vs baseline: 1.2535x; 1.2535x over previous
import jax
import jax.numpy as jnp
from jax.experimental import pallas as pl
from jax.experimental.pallas import tpu as pltpu

NF = 8
FS = 96
D = 768
CD = 64
BM = 512


def _fused_body(x_ref, ssel_ref, e_ref, w_ref, a_ref, b_ref, o_ref):
    xb = x_ref[...]
    scores = jnp.dot(xb, ssel_ref[...], preferred_element_type=jnp.float32)
    m = jnp.max(scores, axis=1, keepdims=True)
    ex = jnp.exp(scores - m)
    p = ex / jnp.sum(ex, axis=1, keepdims=True)
    pe = jnp.dot(p, e_ref[...], preferred_element_type=jnp.float32)
    xp = xb * pe
    xm = xb - xp
    out = jnp.dot(xp, w_ref[...], preferred_element_type=jnp.float32)
    c = jnp.dot(xm, a_ref[...], preferred_element_type=jnp.float32)
    out = out + jnp.dot(c, b_ref[...], preferred_element_type=jnp.float32)
    o_ref[...] = out


def kernel(x, selector_weights, expert_weights, compressor_W, compressed_W):
    batch = x.shape[0]
    w_full = expert_weights.reshape(D, D)
    a = compressor_W.T
    b = compressed_W.T
    fid = jnp.arange(D) // FS
    sel_flat = selector_weights.reshape(D)
    ssel = jnp.zeros((D, NF), x.dtype).at[jnp.arange(D), fid].set(sel_flat)
    e = (jnp.arange(NF)[:, None] == fid[None, :]).astype(x.dtype)

    grid = (batch // BM,)
    out = pl.pallas_call(
        _fused_body,
        grid=grid,
        in_specs=[
            pl.BlockSpec((BM, D), lambda i: (i, 0)),
            pl.BlockSpec((D, NF), lambda i: (0, 0)),
            pl.BlockSpec((NF, D), lambda i: (0, 0)),
            pl.BlockSpec((D, D), lambda i: (0, 0)),
            pl.BlockSpec((D, CD), lambda i: (0, 0)),
            pl.BlockSpec((CD, D), lambda i: (0, 0)),
        ],
        out_specs=pl.BlockSpec((BM, D), lambda i: (i, 0)),
        out_shape=jax.ShapeDtypeStruct((batch, D), x.dtype),
        compiler_params=pltpu.CompilerParams(
            dimension_semantics=("arbitrary",),
        ),
    )(x, ssel, e, w_full, a, b)
    return out
